# Initial kernel scaffold; baseline (speedup 1.0000x reference)
import jax
import jax.numpy as jnp
from jax.experimental import pallas as pl


def kernel(x, field_embeddings):
    raise NotImplementedError("write your pallas kernel here")


# speedup vs baseline: 20.6989x; 20.6989x over previous
import functools

import jax
import jax.numpy as jnp
from jax import lax
from jax.experimental import pallas as pl
from jax.experimental.pallas import tpu as pltpu
from jax.experimental.pallas import tpu_sc as plsc

_N, _M, _K = 100000, 26, 16
_B = 4096
_D = _M * _K
_NC, _NS = 2, 16
_NW = _NC * _NS
_SPW = _B // _NW
_C = 4
_NCH = _SPW // _C
_RPC = _C * _M
_IPW = _SPW * _M


def _permute(v, idx):
    dn = lax.GatherDimensionNumbers(
        offset_dims=(), collapsed_slice_dims=(0,), start_index_map=(0,))
    return lax.gather(v, idx[:, None], dn, slice_sizes=(1,),
                      mode=lax.GatherScatterMode.PROMISE_IN_BOUNDS)


@functools.partial(
    pl.kernel,
    mesh=plsc.VectorSubcoreMesh(core_axis_name="c", subcore_axis_name="s"),
    out_type=jax.ShapeDtypeStruct((_B,), jnp.float32),
    compiler_params=pltpu.CompilerParams(use_tc_tiling_on_sc=False),
    scratch_types=[
        pltpu.VMEM((_IPW,), jnp.int32),
        pltpu.VMEM((2, _RPC, _D), jnp.float32),
        pltpu.VMEM((_SPW * _K,), jnp.float32),
        pltpu.VMEM((_SPW,), jnp.float32),
        pltpu.SemaphoreType.DMA,
        pltpu.SemaphoreType.DMA,
    ],
)
def _ffm_sc(x_hbm, tab_hbm, out_hbm, idx_v, bufs, accs_v, out_v, sem0, sem1):
    wid = lax.axis_index("s") * _NC + lax.axis_index("c")
    lanes = lax.iota(jnp.int32, _K)
    pltpu.sync_copy(x_hbm.at[pl.ds(wid * _IPW, _IPW)], idx_v)
    sems = [sem0, sem1]

    def issue(c, b):
        pltpu.async_copy(
            tab_hbm.at[idx_v.at[pl.ds(c * _RPC, _RPC)]], bufs.at[b], sems[b])

    def wait(b):
        pltpu.make_async_copy(
            tab_hbm.at[idx_v.at[pl.ds(0, _RPC)]], bufs.at[b], sems[b]).wait()

    def compute(c, b):
        buf = bufs.at[b]

        def sample_body(s, _):
            r0 = s * _M
            acc = jnp.zeros((_K,), jnp.float32)
            for j1 in range(_M):
                for j2 in range(j1 + 1, _M):
                    acc = acc + (buf[r0 + j1, pl.ds(j2 * _K, _K)]
                                 * buf[r0 + j2, pl.ds(j1 * _K, _K)])
            for sh in (1, 2, 4, 8):
                acc = acc + _permute(acc, jnp.bitwise_xor(lanes, sh))
            accs_v[pl.ds((c * _C + s) * _K, _K)] = acc
            return 0

        lax.fori_loop(0, _C, sample_body, 0)

    issue(0, 0)

    def outer(i, _):
        for b in range(2):
            c = i * 2 + b

            @pl.when(c + 1 < _NCH)
            def _():
                issue(c + 1, (b + 1) % 2)

            wait(b)
            compute(c, b)
        return 0

    lax.fori_loop(0, _NCH // 2, outer, 0)

    for g in range(_SPW // _K):
        tot = jnp.zeros((_K,), jnp.float32)
        for j in range(_K):
            row = accs_v[pl.ds((g * _K + j) * _K, _K)]
            tot = tot + jnp.where(lanes == j, row, 0.0)
        out_v[pl.ds(g * _K, _K)] = 1.0 / (1.0 + jnp.exp(-tot))
    pltpu.sync_copy(out_v, out_hbm.at[pl.ds(wid * _SPW, _SPW)])


def kernel(x, field_embeddings):
    xf = x.reshape(-1).astype(jnp.int32)
    tab = field_embeddings.reshape(_N, _D)
    return _ffm_sc(xf, tab)


# speedup vs baseline: 102.1674x; 4.9359x over previous
import functools

import jax
import jax.numpy as jnp
from jax import lax
from jax.experimental import pallas as pl
from jax.experimental.pallas import tpu as pltpu
from jax.experimental.pallas import tpu_sc as plsc

_N, _M, _K = 100000, 26, 16
_B = 4096
_D = _M * _K
_DP = 512
_NC, _NS = 2, 16
_NW = _NC * _NS
_SPW = _B // _NW
_C = 4
_NCH = _SPW // _C
_RPC = _C * _M
_IPW = _SPW * _M


def _permute(v, idx):
    dn = lax.GatherDimensionNumbers(
        offset_dims=(), collapsed_slice_dims=(0,), start_index_map=(0,))
    return lax.gather(v, idx[:, None], dn, slice_sizes=(1,),
                      mode=lax.GatherScatterMode.PROMISE_IN_BOUNDS)


@functools.partial(
    pl.kernel,
    mesh=plsc.VectorSubcoreMesh(core_axis_name="c", subcore_axis_name="s"),
    out_type=jax.ShapeDtypeStruct((_B,), jnp.float32),
    compiler_params=pltpu.CompilerParams(use_tc_tiling_on_sc=True),
    scratch_types=[
        pltpu.VMEM((_IPW,), jnp.int32),
        pltpu.VMEM((2, _RPC, _DP), jnp.float32),
        pltpu.VMEM((_SPW * _K,), jnp.float32),
        pltpu.VMEM((_SPW,), jnp.float32),
        pltpu.SemaphoreType.DMA,
        pltpu.SemaphoreType.DMA,
    ],
)
def _ffm_sc(x_hbm, tab_hbm, out_hbm, idx_v, bufs, accs_v, out_v, sem0, sem1):
    wid = lax.axis_index("s") * _NC + lax.axis_index("c")
    lanes = lax.iota(jnp.int32, _K)
    pltpu.sync_copy(x_hbm.at[pl.ds(wid * _IPW, _IPW)], idx_v)
    sems = [sem0, sem1]

    def issue(c, b):
        pltpu.async_copy(
            tab_hbm.at[idx_v.at[pl.ds(c * _RPC, _RPC)]], bufs.at[b], sems[b])

    def wait(b):
        pltpu.make_async_copy(
            tab_hbm.at[idx_v.at[pl.ds(0, _RPC)]], bufs.at[b], sems[b]).wait()

    def compute(c, b):
        buf = bufs.at[b]

        def sample_body(s, _):
            r0 = s * _M
            acc = jnp.zeros((_K,), jnp.float32)
            for j1 in range(_M):
                for j2 in range(j1 + 1, _M):
                    acc = acc + (buf[r0 + j1, pl.ds(j2 * _K, _K)]
                                 * buf[r0 + j2, pl.ds(j1 * _K, _K)])
            for sh in (1, 2, 4, 8):
                acc = acc + _permute(acc, jnp.bitwise_xor(lanes, sh))
            accs_v[pl.ds((c * _C + s) * _K, _K)] = acc
            return 0

        lax.fori_loop(0, _C, sample_body, 0)

    issue(0, 0)

    def outer(i, _):
        for b in range(2):
            c = i * 2 + b

            @pl.when(c + 1 < _NCH)
            def _():
                issue(c + 1, (b + 1) % 2)

            wait(b)
            compute(c, b)
        return 0

    lax.fori_loop(0, _NCH // 2, outer, 0)

    for g in range(_SPW // _K):
        tot = jnp.zeros((_K,), jnp.float32)
        for j in range(_K):
            row = accs_v[pl.ds((g * _K + j) * _K, _K)]
            tot = tot + jnp.where(lanes == j, row, 0.0)
        out_v[pl.ds(g * _K, _K)] = 1.0 / (1.0 + jnp.exp(-tot))
    pltpu.sync_copy(out_v, out_hbm.at[pl.ds(wid * _SPW, _SPW)])


_NB = 2048
_NGRID = -(-_N // _NB)


def _transpose_body(ft_ref, out_ref):
    out_ref[:, : _D] = jnp.transpose(ft_ref[...], (1, 0))


_transpose_tc = pl.pallas_call(
    _transpose_body,
    grid=(_NGRID,),
    in_specs=[pl.BlockSpec((_D, _NB), lambda i: (0, i))],
    out_specs=pl.BlockSpec((_NB, _DP), lambda i: (i, 0)),
    out_shape=jax.ShapeDtypeStruct((_N, _DP), jnp.float32),
)


def kernel(x, field_embeddings):
    xf = x.reshape(-1).astype(jnp.int32)
    ft = jnp.transpose(field_embeddings, (1, 2, 0)).reshape(_D, _N)
    tab = _transpose_tc(ft)
    return _ffm_sc(xf, tab)


# speedup vs baseline: 104.5253x; 1.0231x over previous
import functools

import jax
import jax.numpy as jnp
from jax import lax
from jax.experimental import pallas as pl
from jax.experimental.pallas import tpu as pltpu
from jax.experimental.pallas import tpu_sc as plsc

_N, _M, _K = 100000, 26, 16
_B = 4096
_D = _M * _K
_DP = 512
_NC, _NS = 2, 16
_NW = _NC * _NS
_SPW = _B // _NW
_C = 4
_NCH = _SPW // _C
_RPC = _C * _M
_IPW = _SPW * _M


def _permute(v, idx):
    dn = lax.GatherDimensionNumbers(
        offset_dims=(), collapsed_slice_dims=(0,), start_index_map=(0,))
    return lax.gather(v, idx[:, None], dn, slice_sizes=(1,),
                      mode=lax.GatherScatterMode.PROMISE_IN_BOUNDS)


@functools.partial(
    pl.kernel,
    mesh=plsc.VectorSubcoreMesh(core_axis_name="c", subcore_axis_name="s"),
    out_type=jax.ShapeDtypeStruct((_B,), jnp.float32),
    compiler_params=pltpu.CompilerParams(use_tc_tiling_on_sc=True),
    scratch_types=[
        pltpu.VMEM((_IPW,), jnp.int32),
        pltpu.VMEM((2, _RPC, _DP), jnp.float32),
        pltpu.VMEM((_SPW * _K,), jnp.float32),
        pltpu.VMEM((_SPW,), jnp.float32),
        pltpu.SemaphoreType.DMA,
        pltpu.SemaphoreType.DMA,
    ],
)
def _ffm_sc(x_hbm, tab_hbm, out_hbm, idx_v, bufs, accs_v, out_v, sem0, sem1):
    wid = lax.axis_index("s") * _NC + lax.axis_index("c")
    lanes = lax.iota(jnp.int32, _K)
    pltpu.sync_copy(x_hbm.at[pl.ds(wid * _IPW, _IPW)], idx_v)
    sems = [sem0, sem1]

    def issue(c, b):
        pltpu.async_copy(
            tab_hbm.at[idx_v.at[pl.ds(c * _RPC, _RPC)]], bufs.at[b], sems[b])

    def wait(b):
        pltpu.make_async_copy(
            tab_hbm.at[idx_v.at[pl.ds(0, _RPC)]], bufs.at[b], sems[b]).wait()

    def compute(c, b):
        buf = bufs.at[b]

        def sample_body(s, _):
            r0 = s * _M
            acc = jnp.zeros((_K,), jnp.float32)
            for j1 in range(_M):
                for j2 in range(j1 + 1, _M):
                    acc = acc + (buf[r0 + j1, pl.ds(j2 * _K, _K)]
                                 * buf[r0 + j2, pl.ds(j1 * _K, _K)])
            for sh in (1, 2, 4, 8):
                acc = acc + _permute(acc, jnp.bitwise_xor(lanes, sh))
            accs_v[pl.ds((c * _C + s) * _K, _K)] = acc
            return 0

        lax.fori_loop(0, _C, sample_body, 0)

    issue(0, 0)

    def outer(i, _):
        for b in range(2):
            c = i * 2 + b

            @pl.when(c + 1 < _NCH)
            def _():
                issue(c + 1, (b + 1) % 2)

            wait(b)
            compute(c, b)
        return 0

    lax.fori_loop(0, _NCH // 2, outer, 0)

    for g in range(_SPW // _K):
        tot = jnp.zeros((_K,), jnp.float32)
        for j in range(_K):
            row = accs_v[pl.ds((g * _K + j) * _K, _K)]
            tot = tot + jnp.where(lanes == j, row, 0.0)
        out_v[pl.ds(g * _K, _K)] = 1.0 / (1.0 + jnp.exp(-tot))
    pltpu.sync_copy(out_v, out_hbm.at[pl.ds(wid * _SPW, _SPW)])


_NB = 4096
_NGRID = -(-_N // _NB)


def _transpose_body(ft_ref, out_ref):
    out_ref[:, : _D] = jnp.transpose(ft_ref[...], (1, 0))


_transpose_tc = pl.pallas_call(
    _transpose_body,
    grid=(_NGRID,),
    in_specs=[pl.BlockSpec((_D, _NB), lambda i: (0, i))],
    out_specs=pl.BlockSpec((_NB, _DP), lambda i: (i, 0)),
    out_shape=jax.ShapeDtypeStruct((_N, _DP), jnp.float32),
)


def kernel(x, field_embeddings):
    xf = x.reshape(-1).astype(jnp.int32)
    ft = jnp.transpose(field_embeddings, (1, 2, 0)).reshape(_D, _N)
    tab = _transpose_tc(ft)
    return _ffm_sc(xf, tab)
